# baseline (speedup 1.0000x reference)
import jax
import jax.numpy as jnp
from jax import lax
from jax.experimental import pallas as pl
from jax.experimental.pallas import tpu as pltpu
from jax.experimental.pallas import tpu_sc as plsc

_MAX_DIST = 128
_D = 256
_L = 512
_T_ROWS = 2 * _MAX_DIST + 1

_NC = 2
_NS = 16

_DH = _D // 2
_T16_ROWS = _T_ROWS + 15
_F0_ROWS = 368
_F1_ROWS = 376
_W_ROWS = 1016
_NCLS = 16
_ROWS_PER_CLS = _L // _NCLS


def _body(t16_hbm, f0_hbm, f1_hbm, out_hbm, w_ref, sem):
    wid = lax.axis_index("s") * _NC + lax.axis_index("c")
    h = wid % 2
    c16 = wid // 2

    dh = pl.ds(h * _DH, _DH)

    fills = [
        pltpu.async_copy(f0_hbm.at[:, dh],
                         w_ref.at[pl.ds(0, _F0_ROWS)], sem),
        pltpu.async_copy(t16_hbm.at[c16, :, dh],
                         w_ref.at[pl.ds(_F0_ROWS, _T16_ROWS)], sem),
        pltpu.async_copy(f1_hbm.at[:, dh],
                         w_ref.at[pl.ds(_F0_ROWS + _T16_ROWS, _F1_ROWS)], sem),
    ]
    for f in fills:
        f.wait()

    handles = []
    for t in range(_ROWS_PER_CLS):
        i = c16 + _NCLS * t
        q = (_L - _NCLS) - _NCLS * t
        handles.append(
            pltpu.async_copy(w_ref.at[pl.ds(q, _L)],
                             out_hbm.at[i, :, dh],
                             sem))
    for hd in handles:
        hd.wait()


@jax.jit
def _rpe(table):
    t0 = table[0]
    t256 = table[_T_ROWS - 1]
    base = jnp.concatenate([
        jnp.broadcast_to(t0, (15, _D)),
        table,
        jnp.broadcast_to(t256, (15, _D)),
    ])
    t16 = jnp.stack([
        lax.slice(base, (15 - c_, 0), (15 - c_ + _T16_ROWS, _D))
        for c_ in range(_NCLS)
    ])
    f0 = jnp.broadcast_to(t0, (_F0_ROWS, _D))
    f1 = jnp.broadcast_to(t256, (_F1_ROWS, _D))

    mesh = plsc.VectorSubcoreMesh(core_axis_name="c", subcore_axis_name="s")
    return pl.kernel(
        _body,
        out_type=jax.ShapeDtypeStruct((_L, _L, _D), jnp.float32),
        mesh=mesh,
        scratch_types=[
            pltpu.VMEM((_W_ROWS, _DH), jnp.float32),
            pltpu.SemaphoreType.DMA,
        ],
        compiler_params=pltpu.CompilerParams(use_tc_tiling_on_sc=True),
    )(t16, f0, f1)


def kernel(seq_len, table):
    return _rpe(table)
